# baseline (speedup 1.0000x reference)
import functools

import jax
import jax.numpy as jnp
from jax import lax
from jax.experimental import pallas as pl
from jax.experimental.pallas import tpu as pltpu
from jax.experimental.pallas import tpu_sc as plsc

VOCAB = 1000000
EMBED = 128
PAD_IDX = 2
EPS = 1e-12
B, L = 4096, 200
BL = B * L

NC, NS = 2, 16
NW = NC * NS
RPW = BL // NW
C = 128
NCHUNK = RPW // C
NBUF = 5
CL2 = 2 * L
NV = EMBED // 16
INV_D = 1.0 / EMBED


def _comb_body(pos_ref, seg_ref, out_ref):
    p = pos_ref[...]
    out_ref[0:L, :] = p + seg_ref[0:1, :]
    out_ref[L:CL2, :] = p + seg_ref[1:2, :]


def _vrsqrt(x):
    xb = lax.bitcast_convert_type(x, jnp.int32)
    y = lax.bitcast_convert_type(jnp.int32(0x5F3759DF) - (xb >> 1), jnp.float32)
    half = x * jnp.float32(0.5)
    for _ in range(2):
        y = y * (jnp.float32(1.5) - half * y * y)
    return y


def _tree_add(vs):
    while len(vs) > 1:
        vs = [a + b for a, b in zip(vs[::2], vs[1::2])]
    return vs[0]


@functools.partial(
    pl.kernel,
    mesh=plsc.VectorSubcoreMesh(core_axis_name="c", subcore_axis_name="s"),
    out_type=jax.ShapeDtypeStruct((BL, EMBED), jnp.float32),
    compiler_params=pltpu.CompilerParams(needs_layout_passes=False),
    scratch_types=[
        pltpu.VMEM((NBUF, C), jnp.int32),
        pltpu.VMEM((NBUF, C), jnp.int32),
        pltpu.VMEM((NBUF, C), jnp.int32),
        pltpu.VMEM((NBUF, C, EMBED), jnp.float32),
        pltpu.VMEM((2, 16, 16), jnp.float32),
        pltpu.VMEM((2, EMBED), jnp.float32),
    ] + [pltpu.SemaphoreType.DMA] * (4 * NBUF),
)
def _sc_embed_ln(ids_hbm, tt_hbm, tok_hbm, comb_hbm, gam_hbm, bet_hbm, out_hbm,
                 idx_v, tt_v, cidx_v, rows_v, red_v, gb_v, *sems):
    isem = sems[:NBUF]
    fsem = sems[NBUF:2 * NBUF]
    gsem = sems[2 * NBUF:3 * NBUF]
    osem = sems[3 * NBUF:]
    wid = lax.axis_index("s") * NC + lax.axis_index("c")
    base = wid * RPW
    cbase = wid * CL2

    pltpu.sync_copy(gam_hbm, gb_v.at[0])
    pltpu.sync_copy(bet_hbm, gb_v.at[1])
    gvs = [gb_v[0, pl.ds(16 * j, 16)] for j in range(NV)]
    bvs = [gb_v[1, pl.ds(16 * j, 16)] for j in range(NV)]
    iota16 = lax.iota(jnp.int32, 16)
    zeros16 = jnp.zeros((16,), jnp.int32)
    ones16 = jnp.full((16,), 1, jnp.int32)
    inv_d = jnp.full((16,), INV_D, jnp.float32)

    def start_idx(c, k):
        cb = base + c * C
        pltpu.make_async_copy(ids_hbm.at[pl.ds(cb, C)], idx_v.at[k],
                              isem[k]).start()
        pltpu.make_async_copy(tt_hbm.at[pl.ds(cb, C)], tt_v.at[k],
                              isem[k]).start()

    def wait_idx(c, k):
        cb = base + c * C
        pltpu.make_async_copy(ids_hbm.at[pl.ds(cb, C)], idx_v.at[k],
                              isem[k]).wait()
        pltpu.make_async_copy(tt_hbm.at[pl.ds(cb, C)], tt_v.at[k],
                              isem[k]).wait()

    def start_fill(c, k):
        lbase = lax.rem(c * C, L)
        for i in range(C // 16):
            tt16 = tt_v[k, pl.ds(16 * i, 16)]
            pos16 = lax.rem(lbase + 16 * i + iota16, L)
            cidx_v[k, pl.ds(16 * i, 16)] = tt16 * L + pos16 + cbase
        pltpu.make_async_copy(comb_hbm.at[cidx_v.at[k]], rows_v.at[k],
                              fsem[k]).start()

    def start_tok_add(k):
        pltpu.make_async_copy(tok_hbm.at[idx_v.at[k]], rows_v.at[k],
                              gsem[k]).start(add=True)

    def wait_fill(k):
        pltpu.make_async_copy(comb_hbm.at[cidx_v.at[k]], rows_v.at[k],
                              fsem[k]).wait()

    def wait_tok(k):
        pltpu.make_async_copy(tok_hbm.at[idx_v.at[k]], rows_v.at[k],
                              gsem[k]).wait()

    def out_copy(c, k):
        return pltpu.make_async_copy(
            rows_v.at[k], out_hbm.at[pl.ds(base + c * C, C)], osem[k])

    def ln_chunk(k):
        def group_body(g, carry):
            g16 = g * 16
            for lane in range(16):
                r = g16 + lane
                xs = [rows_v[k, r, pl.ds(16 * j, 16)] for j in range(NV)]
                s = _tree_add(xs)
                ss = _tree_add([x * x for x in xs])
                lcol = jnp.full((16,), lane, jnp.int32)
                plsc.store_scatter(red_v, [zeros16, iota16, lcol], s)
                plsc.store_scatter(red_v, [ones16, iota16, lcol], ss)
            tot = _tree_add([red_v[0, l] for l in range(16)])
            tsq = _tree_add([red_v[1, l] for l in range(16)])
            mean16 = tot * inv_d
            var16 = jnp.maximum(tsq * inv_d - mean16 * mean16,
                                jnp.float32(0.0))
            rstd16 = _vrsqrt(var16 + jnp.float32(EPS))
            for lane in range(16):
                r = g16 + lane
                m = lax.broadcast_in_dim(mean16[lane], (16,), ())
                rs = lax.broadcast_in_dim(rstd16[lane], (16,), ())
                for j in range(NV):
                    x = rows_v[k, r, pl.ds(16 * j, 16)]
                    rows_v[k, r, pl.ds(16 * j, 16)] = (
                        (x - m) * (rs * gvs[j]) + bvs[j])
            return carry

        lax.fori_loop(0, C // 16, group_body, 0)

    start_idx(0, 0)
    start_idx(1, 1)
    start_idx(2, 2)
    wait_idx(0, 0)
    start_fill(0, 0)
    wait_fill(0)
    start_tok_add(0)
    wait_idx(1, 1)
    start_fill(1, 1)

    def block_body(p, carry):
        for k in range(NBUF):
            c = p * NBUF + k
            k1 = (k + 1) % NBUF
            k2 = (k + 2) % NBUF
            k3 = (k + 3) % NBUF

            @pl.when(c + 3 < NCHUNK)
            def _idx():
                start_idx(c + 3, k3)

            @pl.when(c + 2 < NCHUNK)
            def _fill():
                @pl.when(c >= NBUF - 2)
                def _drain():
                    out_copy(0, k2).wait()
                wait_idx(c + 2, k2)
                start_fill(c + 2, k2)

            @pl.when(c + 1 < NCHUNK)
            def _tok():
                wait_fill(k1)
                start_tok_add(k1)

            wait_tok(k)
            ln_chunk(k)
            out_copy(c, k).start()
        return carry

    lax.fori_loop(0, NCHUNK // NBUF, block_body, 0)

    for k in range(NBUF):
        out_copy(0, k).wait()


def kernel(input_ids, token_type_ids, tok_table, pos_table, seg_table, gamma, beta):
    ids = input_ids.reshape(BL).astype(jnp.int32)
    tt = token_type_ids.reshape(BL).astype(jnp.int32)
    comb = pl.pallas_call(
        _comb_body,
        grid=(NW,),
        in_specs=[pl.BlockSpec((L, EMBED), lambda i: (0, 0)),
                  pl.BlockSpec((2, EMBED), lambda i: (0, 0))],
        out_specs=pl.BlockSpec((CL2, EMBED), lambda i: (i, 0)),
        out_shape=jax.ShapeDtypeStruct((NW * CL2, EMBED), jnp.float32),
    )(pos_table[:L], seg_table)
    out = _sc_embed_ln(ids, tt, tok_table, comb, gamma, beta)
    return out.reshape(B, L, EMBED)


# speedup vs baseline: 1.4386x; 1.4386x over previous
import functools

import jax
import jax.numpy as jnp
from jax import lax
from jax.experimental import pallas as pl
from jax.experimental.pallas import tpu as pltpu
from jax.experimental.pallas import tpu_sc as plsc

VOCAB = 1000000
EMBED = 128
PAD_IDX = 2
EPS = 1e-12
B, L = 4096, 200
BL = B * L

NC, NS = 2, 16
NW = NC * NS
NSPLIT = 1
PS = BL // NSPLIT
RPW = PS // NW
C = 128
NCHUNK = RPW // C
NBUF = 5
CL2 = 2 * L


def _comb_body(pos_ref, seg_ref, out_ref):
    p = pos_ref[...]
    out_ref[0:L, :] = p + seg_ref[0:1, :]
    out_ref[L:CL2, :] = p + seg_ref[1:2, :]


def _ln_body(x_ref, g_ref, b_ref, o_ref):
    x = x_ref[...]
    mean = jnp.mean(x, axis=1, keepdims=True)
    xc = x - mean
    var = jnp.mean(xc * xc, axis=1, keepdims=True)
    o_ref[...] = xc * lax.rsqrt(var + EPS) * g_ref[...] + b_ref[...]


@functools.partial(
    pl.kernel,
    mesh=plsc.VectorSubcoreMesh(core_axis_name="c", subcore_axis_name="s"),
    out_type=jax.ShapeDtypeStruct((PS, EMBED), jnp.float32),
    compiler_params=pltpu.CompilerParams(needs_layout_passes=False),
    scratch_types=[
        pltpu.VMEM((NBUF, C), jnp.int32),
        pltpu.VMEM((NBUF, C), jnp.int32),
        pltpu.VMEM((NBUF, C), jnp.int32),
        pltpu.VMEM((NBUF, C, EMBED), jnp.float32),
        pltpu.VMEM_SHARED((CL2, EMBED), jnp.float32),
    ] + [pltpu.SemaphoreType.DMA] * (4 * NBUF),
)
def _sc_gather_sum(ids_hbm, tt_hbm, tok_hbm, comb_hbm, out_hbm,
                   idx_v, tt_v, cidx_v, rows_v, comb_sh, *sems):
    isem = sems[:NBUF]
    fsem = sems[NBUF:2 * NBUF]
    gsem = sems[2 * NBUF:3 * NBUF]
    osem = sems[3 * NBUF:]
    sid = lax.axis_index("s")
    wid = sid * NC + lax.axis_index("c")
    base = wid * RPW

    @pl.when(sid == 0)
    def _stage_comb():
        pltpu.sync_copy(comb_hbm, comb_sh)
    plsc.subcore_barrier()

    def start_idx(c, k):
        cb = base + c * C
        pltpu.make_async_copy(ids_hbm.at[pl.ds(cb, C)], idx_v.at[k],
                              isem[k]).start()
        pltpu.make_async_copy(tt_hbm.at[pl.ds(cb, C)], tt_v.at[k],
                              isem[k]).start()

    def wait_idx(c, k):
        cb = base + c * C
        pltpu.make_async_copy(ids_hbm.at[pl.ds(cb, C)], idx_v.at[k],
                              isem[k]).wait()
        pltpu.make_async_copy(tt_hbm.at[pl.ds(cb, C)], tt_v.at[k],
                              isem[k]).wait()

    def start_fill(c, k):
        lbase = lax.rem(c * C, L)
        for i in range(C // 16):
            tt16 = tt_v[k, pl.ds(16 * i, 16)]
            pos16 = lax.rem(lbase + 16 * i + lax.iota(jnp.int32, 16), L)
            cidx_v[k, pl.ds(16 * i, 16)] = tt16 * L + pos16
        pltpu.make_async_copy(comb_sh.at[cidx_v.at[k]], rows_v.at[k],
                              fsem[k]).start()

    def start_tok_add(k):
        pltpu.make_async_copy(tok_hbm.at[idx_v.at[k]], rows_v.at[k],
                              gsem[k]).start(add=True)

    def wait_fill(k):
        pltpu.make_async_copy(comb_sh.at[cidx_v.at[k]], rows_v.at[k],
                              fsem[k]).wait()

    def wait_tok(k):
        pltpu.make_async_copy(tok_hbm.at[idx_v.at[k]], rows_v.at[k],
                              gsem[k]).wait()

    def out_copy(c, k):
        return pltpu.make_async_copy(
            rows_v.at[k], out_hbm.at[pl.ds(base + c * C, C)], osem[k])

    start_idx(0, 0)
    start_idx(1, 1)
    start_idx(2, 2)
    wait_idx(0, 0)
    start_fill(0, 0)
    wait_fill(0)
    start_tok_add(0)
    wait_idx(1, 1)
    start_fill(1, 1)

    def block_body(p, carry):
        for k in range(NBUF):
            c = p * NBUF + k
            k1 = (k + 1) % NBUF
            k2 = (k + 2) % NBUF
            k3 = (k + 3) % NBUF

            @pl.when(c + 3 < NCHUNK)
            def _idx():
                start_idx(c + 3, k3)

            @pl.when(c + 2 < NCHUNK)
            def _fill():
                @pl.when(c >= NBUF - 2)
                def _drain():
                    out_copy(0, k2).wait()
                wait_idx(c + 2, k2)
                start_fill(c + 2, k2)

            @pl.when(c + 1 < NCHUNK)
            def _tok():
                wait_fill(k1)
                start_tok_add(k1)

            wait_tok(k)
            out_copy(c, k).start()
        return carry

    lax.fori_loop(0, NCHUNK // NBUF, block_body, 0)

    for k in range(NBUF):
        out_copy(0, k).wait()


RB = 4096


def kernel(input_ids, token_type_ids, tok_table, pos_table, seg_table, gamma, beta):
    ids = input_ids.reshape(BL).astype(jnp.int32)
    tt = token_type_ids.reshape(BL).astype(jnp.int32)
    comb = pl.pallas_call(
        _comb_body,
        out_shape=jax.ShapeDtypeStruct((CL2, EMBED), jnp.float32),
    )(pos_table[:L], seg_table)
    ln = pl.pallas_call(
        _ln_body,
        grid=(PS // RB,),
        in_specs=[pl.BlockSpec((RB, EMBED), lambda i: (i, 0)),
                  pl.BlockSpec((1, EMBED), lambda i: (0, 0)),
                  pl.BlockSpec((1, EMBED), lambda i: (0, 0))],
        out_specs=pl.BlockSpec((RB, EMBED), lambda i: (i, 0)),
        out_shape=jax.ShapeDtypeStruct((PS, EMBED), jnp.float32),
    )
    g2 = gamma.reshape(1, EMBED)
    b2 = beta.reshape(1, EMBED)
    summed = [_sc_gather_sum(ids[i * PS:(i + 1) * PS],
                             tt[i * PS:(i + 1) * PS], tok_table, comb)
              for i in range(NSPLIT)]
    parts = [ln(s, g2, b2) for s in summed]
    return jnp.concatenate(parts, axis=0).reshape(B, L, EMBED)


# speedup vs baseline: 1.5678x; 1.0898x over previous
import functools

import jax
import jax.numpy as jnp
from jax import lax
from jax.experimental import pallas as pl
from jax.experimental.pallas import tpu as pltpu
from jax.experimental.pallas import tpu_sc as plsc

VOCAB = 1000000
EMBED = 128
PAD_IDX = 2
EPS = 1e-12
B, L = 4096, 200
BL = B * L

NC, NS = 2, 16
NW = NC * NS
NSPLIT = 1
PS = BL // NSPLIT
RPW = PS // NW
C = 128
NCHUNK = RPW // C
NBUF = 5
CL2 = 2 * L


def _comb_body(pos_ref, seg_ref, out_ref):
    p = pos_ref[...]
    out_ref[0:L, :] = p + seg_ref[0:1, :]
    out_ref[L:CL2, :] = p + seg_ref[1:2, :]


def _ln_body(x_ref, g_ref, b_ref, o_ref):
    x = x_ref[...]
    mean = jnp.mean(x, axis=1, keepdims=True)
    xc = x - mean
    var = jnp.mean(xc * xc, axis=1, keepdims=True)
    o_ref[...] = xc * lax.rsqrt(var + EPS) * g_ref[...] + b_ref[...]


@functools.partial(
    pl.kernel,
    mesh=plsc.VectorSubcoreMesh(core_axis_name="c", subcore_axis_name="s"),
    out_type=jax.ShapeDtypeStruct((PS, EMBED), jnp.float32),
    compiler_params=pltpu.CompilerParams(needs_layout_passes=False),
    scratch_types=[
        pltpu.VMEM((NBUF, C), jnp.int32),
        pltpu.VMEM((NBUF, C), jnp.int32),
        pltpu.VMEM((NBUF, C), jnp.int32),
        pltpu.VMEM((NBUF, C, EMBED), jnp.float32),
        pltpu.VMEM_SHARED((CL2, EMBED), jnp.float32),
    ] + [pltpu.SemaphoreType.DMA] * (4 * NBUF),
)
def _sc_gather_sum(ids_hbm, tt_hbm, tok_hbm, comb_hbm, out_hbm,
                   idx_v, tt_v, cidx_v, rows_v, comb_sh, *sems):
    isem = sems[:NBUF]
    fsem = sems[NBUF:2 * NBUF]
    gsem = sems[2 * NBUF:3 * NBUF]
    osem = sems[3 * NBUF:]
    sid = lax.axis_index("s")
    wid = sid * NC + lax.axis_index("c")
    base = wid * RPW

    @pl.when(sid == 0)
    def _stage_comb():
        pltpu.sync_copy(comb_hbm, comb_sh)
    plsc.subcore_barrier()

    def start_idx(c, k):
        cb = base + c * C
        pltpu.make_async_copy(ids_hbm.at[pl.ds(cb, C)], idx_v.at[k],
                              isem[k]).start()
        pltpu.make_async_copy(tt_hbm.at[pl.ds(cb, C)], tt_v.at[k],
                              isem[k]).start()

    def wait_idx(c, k):
        cb = base + c * C
        pltpu.make_async_copy(ids_hbm.at[pl.ds(cb, C)], idx_v.at[k],
                              isem[k]).wait()
        pltpu.make_async_copy(tt_hbm.at[pl.ds(cb, C)], tt_v.at[k],
                              isem[k]).wait()

    def start_fill(c, k):
        lbase = lax.rem(c * C, L)
        for i in range(C // 16):
            tt16 = tt_v[k, pl.ds(16 * i, 16)]
            pos16 = lax.rem(lbase + 16 * i + lax.iota(jnp.int32, 16), L)
            cidx_v[k, pl.ds(16 * i, 16)] = tt16 * L + pos16
        pltpu.make_async_copy(comb_sh.at[cidx_v.at[k]], rows_v.at[k],
                              fsem[k]).start()

    def start_tok_add(k):
        pltpu.make_async_copy(tok_hbm.at[idx_v.at[k]], rows_v.at[k],
                              gsem[k]).start(add=True)

    def wait_fill(k):
        pltpu.make_async_copy(comb_sh.at[cidx_v.at[k]], rows_v.at[k],
                              fsem[k]).wait()

    def wait_tok(k):
        pltpu.make_async_copy(tok_hbm.at[idx_v.at[k]], rows_v.at[k],
                              gsem[k]).wait()

    def out_copy(c, k):
        return pltpu.make_async_copy(
            rows_v.at[k], out_hbm.at[pl.ds(base + c * C, C)], osem[k])

    start_idx(0, 0)
    start_idx(1, 1)
    start_idx(2, 2)
    wait_idx(0, 0)
    start_fill(0, 0)
    wait_fill(0)
    start_tok_add(0)
    wait_idx(1, 1)
    start_fill(1, 1)

    def block_body(p, carry):
        for k in range(NBUF):
            c = p * NBUF + k
            k1 = (k + 1) % NBUF
            k2 = (k + 2) % NBUF
            k3 = (k + 3) % NBUF

            @pl.when(c + 3 < NCHUNK)
            def _idx():
                start_idx(c + 3, k3)

            @pl.when(c + 2 < NCHUNK)
            def _fill():
                @pl.when(c >= NBUF - 2)
                def _drain():
                    out_copy(0, k2).wait()
                wait_idx(c + 2, k2)
                start_fill(c + 2, k2)

            @pl.when(c + 1 < NCHUNK)
            def _tok():
                wait_fill(k1)
                start_tok_add(k1)

            wait_tok(k)
            out_copy(c, k).start()
        return carry

    lax.fori_loop(0, NCHUNK // NBUF, block_body, 0)

    for k in range(NBUF):
        out_copy(0, k).wait()


RB = 8192


def kernel(input_ids, token_type_ids, tok_table, pos_table, seg_table, gamma, beta):
    ids = input_ids.reshape(BL).astype(jnp.int32)
    tt = token_type_ids.reshape(BL).astype(jnp.int32)
    comb = pl.pallas_call(
        _comb_body,
        out_shape=jax.ShapeDtypeStruct((CL2, EMBED), jnp.float32),
    )(pos_table[:L], seg_table)
    ln = pl.pallas_call(
        _ln_body,
        grid=(PS // RB,),
        in_specs=[pl.BlockSpec((RB, EMBED), lambda i: (i, 0)),
                  pl.BlockSpec((1, EMBED), lambda i: (0, 0)),
                  pl.BlockSpec((1, EMBED), lambda i: (0, 0))],
        out_specs=pl.BlockSpec((RB, EMBED), lambda i: (i, 0)),
        out_shape=jax.ShapeDtypeStruct((PS, EMBED), jnp.float32),
    )
    g2 = gamma.reshape(1, EMBED)
    b2 = beta.reshape(1, EMBED)
    summed = [_sc_gather_sum(ids[i * PS:(i + 1) * PS],
                             tt[i * PS:(i + 1) * PS], tok_table, comb)
              for i in range(NSPLIT)]
    parts = [ln(s, g2, b2) for s in summed]
    return jnp.concatenate(parts, axis=0).reshape(B, L, EMBED)


# speedup vs baseline: 1.6362x; 1.0436x over previous
import functools

import jax
import jax.numpy as jnp
from jax import lax
from jax.experimental import pallas as pl
from jax.experimental.pallas import tpu as pltpu
from jax.experimental.pallas import tpu_sc as plsc

VOCAB = 1000000
EMBED = 128
PAD_IDX = 2
EPS = 1e-12
B, L = 4096, 200
BL = B * L

NC, NS = 2, 16
NW = NC * NS
NSPLIT = 1
PS = BL // NSPLIT
RPW = PS // NW
C = 128
NCHUNK = RPW // C
NBUF = 5
CL2 = 2 * L


def _comb_body(pos_ref, seg_ref, out_ref):
    p = pos_ref[...]
    out_ref[0:L, :] = p + seg_ref[0:1, :]
    out_ref[L:CL2, :] = p + seg_ref[1:2, :]


def _ln_body(x_ref, g_ref, b_ref, o_ref):
    x = x_ref[...]
    mean = jnp.mean(x, axis=1, keepdims=True)
    xc = x - mean
    var = jnp.mean(xc * xc, axis=1, keepdims=True)
    o_ref[...] = xc * lax.rsqrt(var + EPS) * g_ref[...] + b_ref[...]


@functools.partial(
    pl.kernel,
    mesh=plsc.VectorSubcoreMesh(core_axis_name="c", subcore_axis_name="s"),
    out_type=jax.ShapeDtypeStruct((PS, EMBED), jnp.float32),
    compiler_params=pltpu.CompilerParams(needs_layout_passes=False),
    scratch_types=[
        pltpu.VMEM((NBUF, C), jnp.int32),
        pltpu.VMEM((NBUF, C), jnp.int32),
        pltpu.VMEM((NBUF, C), jnp.int32),
        pltpu.VMEM((NBUF, C, EMBED), jnp.float32),
        pltpu.VMEM_SHARED((CL2, EMBED), jnp.float32),
    ] + [pltpu.SemaphoreType.DMA] * (4 * NBUF),
)
def _sc_gather_sum(ids_hbm, tt_hbm, tok_hbm, comb_hbm, out_hbm,
                   idx_v, tt_v, cidx_v, rows_v, comb_sh, *sems):
    isem = sems[:NBUF]
    fsem = sems[NBUF:2 * NBUF]
    gsem = sems[2 * NBUF:3 * NBUF]
    osem = sems[3 * NBUF:]
    sid = lax.axis_index("s")
    wid = sid * NC + lax.axis_index("c")
    base = wid * RPW

    @pl.when(sid == 0)
    def _stage_comb():
        pltpu.sync_copy(comb_hbm, comb_sh)
    plsc.subcore_barrier()

    def start_idx(c, k):
        cb = base + c * C
        pltpu.make_async_copy(ids_hbm.at[pl.ds(cb, C)], idx_v.at[k],
                              isem[k]).start()
        pltpu.make_async_copy(tt_hbm.at[pl.ds(cb, C)], tt_v.at[k],
                              isem[k]).start()

    def wait_idx(c, k):
        cb = base + c * C
        pltpu.make_async_copy(ids_hbm.at[pl.ds(cb, C)], idx_v.at[k],
                              isem[k]).wait()
        pltpu.make_async_copy(tt_hbm.at[pl.ds(cb, C)], tt_v.at[k],
                              isem[k]).wait()

    def start_fill(c, k):
        lbase = lax.rem(c * C, L)
        for i in range(C // 16):
            tt16 = tt_v[k, pl.ds(16 * i, 16)]
            pos16 = lax.rem(lbase + 16 * i + lax.iota(jnp.int32, 16), L)
            cidx_v[k, pl.ds(16 * i, 16)] = tt16 * L + pos16
        pltpu.make_async_copy(comb_sh.at[cidx_v.at[k]], rows_v.at[k],
                              fsem[k]).start()

    def start_tok_add(k):
        pltpu.make_async_copy(tok_hbm.at[idx_v.at[k]], rows_v.at[k],
                              gsem[k]).start(add=True)

    def wait_fill(k):
        pltpu.make_async_copy(comb_sh.at[cidx_v.at[k]], rows_v.at[k],
                              fsem[k]).wait()

    def wait_tok(k):
        pltpu.make_async_copy(tok_hbm.at[idx_v.at[k]], rows_v.at[k],
                              gsem[k]).wait()

    def out_copy(c, k):
        return pltpu.make_async_copy(
            rows_v.at[k], out_hbm.at[pl.ds(base + c * C, C)], osem[k])

    start_idx(0, 0)
    start_idx(1, 1)
    start_idx(2, 2)
    wait_idx(0, 0)
    start_fill(0, 0)
    wait_fill(0)
    start_tok_add(0)
    wait_idx(1, 1)
    start_fill(1, 1)

    def block_body(p, carry):
        for k in range(NBUF):
            c = p * NBUF + k
            k1 = (k + 1) % NBUF
            k2 = (k + 2) % NBUF
            k3 = (k + 3) % NBUF

            @pl.when(c + 3 < NCHUNK)
            def _idx():
                start_idx(c + 3, k3)

            @pl.when(c + 2 < NCHUNK)
            def _fill():
                @pl.when(c >= NBUF - 2)
                def _drain():
                    out_copy(0, k2).wait()
                wait_idx(c + 2, k2)
                start_fill(c + 2, k2)

            @pl.when(c + 1 < NCHUNK)
            def _tok():
                wait_fill(k1)
                start_tok_add(k1)

            wait_tok(k)
            out_copy(c, k).start()
        return carry

    lax.fori_loop(0, NCHUNK // NBUF, block_body, 0)

    for k in range(NBUF):
        out_copy(0, k).wait()


RB = 16384


def kernel(input_ids, token_type_ids, tok_table, pos_table, seg_table, gamma, beta):
    ids = input_ids.reshape(BL).astype(jnp.int32)
    tt = token_type_ids.reshape(BL).astype(jnp.int32)
    comb = pl.pallas_call(
        _comb_body,
        out_shape=jax.ShapeDtypeStruct((CL2, EMBED), jnp.float32),
    )(pos_table[:L], seg_table)
    ln = pl.pallas_call(
        _ln_body,
        grid=(PS // RB,),
        in_specs=[pl.BlockSpec((RB, EMBED), lambda i: (i, 0)),
                  pl.BlockSpec((1, EMBED), lambda i: (0, 0)),
                  pl.BlockSpec((1, EMBED), lambda i: (0, 0))],
        out_specs=pl.BlockSpec((RB, EMBED), lambda i: (i, 0)),
        out_shape=jax.ShapeDtypeStruct((PS, EMBED), jnp.float32),
    )
    g2 = gamma.reshape(1, EMBED)
    b2 = beta.reshape(1, EMBED)
    summed = [_sc_gather_sum(ids[i * PS:(i + 1) * PS],
                             tt[i * PS:(i + 1) * PS], tok_table, comb)
              for i in range(NSPLIT)]
    parts = [ln(s, g2, b2) for s in summed]
    return jnp.concatenate(parts, axis=0).reshape(B, L, EMBED)
